# baseline (speedup 1.0000x reference)
import jax
import jax.numpy as jnp
from jax.experimental import pallas as pl
from jax.experimental.pallas import tpu as pltpu

N_TOK = 32768
D_IN = 768
D_HID = 64
D_OUT = 768

_RAMP = [512, 512, 1024, 2048]
TILES = _RAMP + [4096] * 6 + _RAMP[::-1]
assert sum(TILES) == N_TOK
OFFS = [sum(TILES[:i]) for i in range(len(TILES))]
G = len(TILES)
MAXT = max(TILES)
K = 2


def _mlp_manual(x_hbm, w1t_ref, b1_ref, w2t_ref, b2_ref, out_hbm,
                xbuf, obuf, insem, outsem):
    def in_copy(i):
        slot = i % K
        return pltpu.make_async_copy(
            x_hbm.at[pl.ds(OFFS[i], TILES[i]), :],
            xbuf.at[slot, pl.ds(0, TILES[i]), :], insem.at[slot])

    def out_copy(i):
        slot = i % K
        return pltpu.make_async_copy(
            obuf.at[slot, pl.ds(0, TILES[i]), :],
            out_hbm.at[pl.ds(OFFS[i], TILES[i]), :], outsem.at[slot])

    for i in range(K - 1):
        in_copy(i).start()

    w1 = w1t_ref[...].astype(jnp.bfloat16)
    w2 = w2t_ref[...].astype(jnp.bfloat16)
    b1v = b1_ref[...]
    b2v = b2_ref[...]

    for i in range(G):
        in_copy(i).wait()
        if i + K - 1 < G:
            in_copy(i + K - 1).start()
        if i >= K:
            out_copy(i - K).wait()
        t = TILES[i]
        xb = xbuf[i % K, :t, :].astype(jnp.bfloat16)
        h = jnp.maximum(
            jnp.dot(xb, w1, preferred_element_type=jnp.float32) + b1v, 0.0)
        obuf[i % K, :t, :] = jnp.dot(h.astype(jnp.bfloat16), w2,
                                     preferred_element_type=jnp.float32) + b2v
        out_copy(i).start()

    for i in range(max(G - K, 0), G):
        out_copy(i).wait()


def kernel(x, W1, b1, W2, b2):
    w1t = W1.T
    w2t = W2.T
    b1r = b1.reshape(1, D_HID)
    b2r = b2.reshape(1, D_OUT)

    out = pl.pallas_call(
        _mlp_manual,
        in_specs=[
            pl.BlockSpec(memory_space=pl.ANY),
            pl.BlockSpec((D_IN, D_HID), lambda: (0, 0)),
            pl.BlockSpec((1, D_HID), lambda: (0, 0)),
            pl.BlockSpec((D_HID, D_OUT), lambda: (0, 0)),
            pl.BlockSpec((1, D_OUT), lambda: (0, 0)),
        ],
        out_specs=pl.BlockSpec(memory_space=pl.ANY),
        out_shape=jax.ShapeDtypeStruct((N_TOK, D_OUT), jnp.float32),
        scratch_shapes=[
            pltpu.VMEM((K, MAXT, D_IN), jnp.float32),
            pltpu.VMEM((K, MAXT, D_OUT), jnp.float32),
            pltpu.SemaphoreType.DMA((K,)),
            pltpu.SemaphoreType.DMA((K,)),
        ],
        compiler_params=pltpu.CompilerParams(
            vmem_limit_bytes=128 * 1024 * 1024,
        ),
    )(x, w1t, b1r, w2t, b2r)

    aux = jnp.zeros((), dtype=jnp.float32)
    return (out, aux)


# speedup vs baseline: 1.1349x; 1.1349x over previous
import jax
import jax.numpy as jnp
from jax.experimental import pallas as pl
from jax.experimental.pallas import tpu as pltpu

N_TOK = 32768
D_IN = 768
D_HID = 64
D_OUT = 768

_RAMP = [512, 512, 1024]
TILES = _RAMP + [2048] * 14 + _RAMP[::-1]
assert sum(TILES) == N_TOK
OFFS = [sum(TILES[:i]) for i in range(len(TILES))]
G = len(TILES)
MAXT = max(TILES)
K = 4


def _mlp_manual(x_hbm, w1t_ref, b1_ref, w2t_ref, b2_ref, out_hbm,
                xbuf, obuf, insem, outsem):
    def in_copy(i):
        slot = i % K
        return pltpu.make_async_copy(
            x_hbm.at[pl.ds(OFFS[i], TILES[i]), :],
            xbuf.at[slot, pl.ds(0, TILES[i]), :], insem.at[slot])

    def out_copy(i):
        slot = i % K
        return pltpu.make_async_copy(
            obuf.at[slot, pl.ds(0, TILES[i]), :],
            out_hbm.at[pl.ds(OFFS[i], TILES[i]), :], outsem.at[slot])

    for i in range(K - 1):
        in_copy(i).start()

    w1 = w1t_ref[...].astype(jnp.bfloat16)
    w2 = w2t_ref[...].astype(jnp.bfloat16)
    b1v = b1_ref[...]
    b2v = b2_ref[...]

    for i in range(G):
        in_copy(i).wait()
        if i + K - 1 < G:
            in_copy(i + K - 1).start()
        if i >= K:
            out_copy(i - K).wait()
        t = TILES[i]
        xb = xbuf[i % K, :t, :].astype(jnp.bfloat16)
        h = jnp.maximum(
            jnp.dot(xb, w1, preferred_element_type=jnp.float32) + b1v, 0.0)
        obuf[i % K, :t, :] = jnp.dot(h.astype(jnp.bfloat16), w2,
                                     preferred_element_type=jnp.float32) + b2v
        out_copy(i).start()

    for i in range(max(G - K, 0), G):
        out_copy(i).wait()


def kernel(x, W1, b1, W2, b2):
    w1t = W1.T
    w2t = W2.T
    b1r = b1.reshape(1, D_HID)
    b2r = b2.reshape(1, D_OUT)

    out = pl.pallas_call(
        _mlp_manual,
        in_specs=[
            pl.BlockSpec(memory_space=pl.ANY),
            pl.BlockSpec((D_IN, D_HID), lambda: (0, 0)),
            pl.BlockSpec((1, D_HID), lambda: (0, 0)),
            pl.BlockSpec((D_HID, D_OUT), lambda: (0, 0)),
            pl.BlockSpec((1, D_OUT), lambda: (0, 0)),
        ],
        out_specs=pl.BlockSpec(memory_space=pl.ANY),
        out_shape=jax.ShapeDtypeStruct((N_TOK, D_OUT), jnp.float32),
        scratch_shapes=[
            pltpu.VMEM((K, MAXT, D_IN), jnp.float32),
            pltpu.VMEM((K, MAXT, D_OUT), jnp.float32),
            pltpu.SemaphoreType.DMA((K,)),
            pltpu.SemaphoreType.DMA((K,)),
        ],
        compiler_params=pltpu.CompilerParams(
            vmem_limit_bytes=128 * 1024 * 1024,
        ),
    )(x, w1t, b1r, w2t, b2r)

    aux = jnp.zeros((), dtype=jnp.float32)
    return (out, aux)
